# Initial kernel scaffold; baseline (speedup 1.0000x reference)
import jax
import jax.numpy as jnp
from jax.experimental import pallas as pl


def kernel(edge_feats, phase, edge_index, W1, b1, W2, b2, Wn, bn, lstm_kernel, lstm_rkernel, lstm_bias, Wa, Wq1, bq1, Wq2, bq2):
    raise NotImplementedError("write your pallas kernel here")


# speedup vs baseline: 8.8237x; 8.8237x over previous
import jax
import jax.numpy as jnp
from jax import lax
from jax.experimental import pallas as pl
from jax.experimental.pallas import tpu as pltpu
from jax.experimental.pallas import tpu_sc as plsc

_N = 100000
_E = 3200000
_NC, _NS, _L = 2, 16, 16
_NW = _NC * _NS
_EPW = _E // _NW
_KA = 800
_KC = 400
_NP = 100352
_RPS = _NP // _NS
_ZR = 392
_f32 = jnp.float32
_i32 = jnp.int32

_sc_mesh = plsc.VectorSubcoreMesh(
    core_axis_name="c", subcore_axis_name="s", num_cores=_NC, num_subcores=_NS)
_sc_params = pltpu.CompilerParams(use_tc_tiling_on_sc=False, needs_layout_passes=False)


def _mlp_body(x_ref, w1_ref, b1_ref, w2_ref, b2_ref, o_ref):
    x = x_ref[...]
    z1 = jax.nn.sigmoid(
        jnp.dot(x, w1_ref[...], preferred_element_type=_f32) + b1_ref[...])
    z2 = jnp.dot(z1, w2_ref[...], preferred_element_type=_f32) + b2_ref[...]
    o_ref[...] = jnp.maximum(z2, 0.0)


def _edge_mlp(ef, w1p, b1p, w2p, b2p):
    be = 6400
    return pl.pallas_call(
        _mlp_body,
        grid=(_E // be,),
        in_specs=[
            pl.BlockSpec((be, 10), lambda i: (i, 0)),
            pl.BlockSpec((10, 16), lambda i: (0, 0)),
            pl.BlockSpec((1, 16), lambda i: (0, 0)),
            pl.BlockSpec((16, 16), lambda i: (0, 0)),
            pl.BlockSpec((1, 16), lambda i: (0, 0)),
        ],
        out_specs=pl.BlockSpec((be, 16), lambda i: (i, 0)),
        out_shape=jax.ShapeDtypeStruct((_E, 16), _f32),
    )(ef, w1p, b1p, w2p, b2p)


def _zero_acc(zbuf, acc, s):
    zero16 = jnp.zeros((_L,), _f32)

    def zb(i, _):
        zbuf[i, :] = zero16
        return 0
    lax.fori_loop(0, _ZR, zb, 0)

    def zacc(j, _):
        pltpu.sync_copy(zbuf, acc.at[pl.ds(s * _RPS + j * _ZR, _ZR)])
        return 0
    lax.fori_loop(0, _RPS // _ZR, zacc, 0)


def _sc_scatter_body(eh_hbm, dst_hbm, agg_hbm, dstbuf, valbuf, zbuf, acc):
    c = lax.axis_index("c")
    s = lax.axis_index("s")
    wid = c * _NS + s
    _zero_acc(zbuf, acc, s)
    plsc.subcore_barrier()

    def chunk(t, _):
        base = wid * _EPW + t * _KA
        pltpu.sync_copy(dst_hbm.at[pl.ds(base, _KA)], dstbuf)
        pltpu.sync_copy(eh_hbm.at[pl.ds(base, _KA)], valbuf)
        pltpu.sync_copy(valbuf, acc.at[dstbuf], add=True)
        return 0
    lax.fori_loop(0, _EPW // _KA, chunk, 0)
    plsc.subcore_barrier()
    pltpu.sync_copy(acc.at[pl.ds(s * _RPS, _RPS)],
                    agg_hbm.at[c, pl.ds(s * _RPS, _RPS)])


_sc_scatter = pl.kernel(
    _sc_scatter_body,
    out_type=jax.ShapeDtypeStruct((_NC, _NP, 16), _f32),
    mesh=_sc_mesh,
    compiler_params=_sc_params,
    scratch_types=[
        pltpu.VMEM((_KA,), _i32),
        pltpu.VMEM((_KA, 16), _f32),
        pltpu.VMEM((_ZR, 16), _f32),
        pltpu.VMEM_SHARED((_NP, 16), _f32),
    ],
)


def _mid_body(a0_ref, a1_ref, ph_ref, wn_ref, bn_ref, ki_ref, bi_ref,
              kc_ref, bc_ref, ko_ref, bo_ref, wa1_ref, wa2_ref,
              nv_ref, tab_ref, c_ref):
    agg = a0_ref[0] + a1_ref[0]
    ph = ph_ref[...]
    lane = lax.broadcasted_iota(_i32, agg.shape, 1)
    node_in = agg + (lane == ph + 9).astype(_f32)
    nv = jnp.dot(node_in, wn_ref[...], preferred_element_type=_f32) + bn_ref[...]
    i_g = jax.nn.sigmoid(
        jnp.dot(nv, ki_ref[...], preferred_element_type=_f32) + bi_ref[...])
    c_bar = jnp.tanh(
        jnp.dot(nv, kc_ref[...], preferred_element_type=_f32) + bc_ref[...])
    o_g = jax.nn.sigmoid(
        jnp.dot(nv, ko_ref[...], preferred_element_type=_f32) + bo_ref[...])
    cc = i_g * c_bar
    h = o_g * jnp.tanh(cc)
    a = jnp.sum(h * wa1_ref[...], axis=1, keepdims=True)
    b = jnp.sum(h * wa2_ref[...], axis=1, keepdims=True)
    nv_ref[...] = nv
    c_ref[...] = cc
    tab_ref[...] = jnp.concatenate([a, b, h[:, :14]], axis=1)


def _tc_mid(agg, ph2, wn16, bn16, ki, bi, kc, bc, ko, bo, wa1, wa2):
    r = 2000
    full = lambda shape: pl.BlockSpec(shape, lambda i: tuple(0 for _ in shape))
    return pl.pallas_call(
        _mid_body,
        grid=(_N // r,),
        in_specs=[
            pl.BlockSpec((1, r, 16), lambda i: (0, i, 0)),
            pl.BlockSpec((1, r, 16), lambda i: (1, i, 0)),
            pl.BlockSpec((r, 1), lambda i: (i, 0)),
            full((16, 16)), full((1, 16)),
            full((16, 16)), full((1, 16)),
            full((16, 16)), full((1, 16)),
            full((16, 16)), full((1, 16)),
            full((1, 16)), full((1, 16)),
        ],
        out_specs=[
            pl.BlockSpec((r, 16), lambda i: (i, 0)),
            pl.BlockSpec((r, 16), lambda i: (i, 0)),
            pl.BlockSpec((r, 16), lambda i: (i, 0)),
        ],
        out_shape=[
            jax.ShapeDtypeStruct((_N, 16), _f32),
            jax.ShapeDtypeStruct((_N, 16), _f32),
            jax.ShapeDtypeStruct((_N, 16), _f32),
        ],
    )(agg, agg, ph2, wn16, bn16, ki, bi, kc, bc, ko, bo, wa1, wa2)


def _sc_attn_body(tab_hbm, src_hbm, dst_hbm, att_hbm, srcbuf, dstbuf, srcrows, dstrows,
                  outbuf, zbuf, acc, sem1, sem2):
    c = lax.axis_index("c")
    s = lax.axis_index("s")
    wid = c * _NS + s
    iota16 = lax.iota(_i32, _L)
    zero16 = jnp.zeros((_L,), _f32)
    _zero_acc(zbuf, acc, s)

    c15 = jnp.full((_L,), 15, _i32)

    def zo(g, _):
        plsc.store_scatter(outbuf, [g * _L + iota16, c15], zero16)
        return 0
    lax.fori_loop(0, _KC // _L, zo, 0)
    plsc.subcore_barrier()

    def chunk(t, _):
        base = wid * _EPW + t * _KC
        pltpu.sync_copy(src_hbm.at[pl.ds(base, _KC)], srcbuf)
        pltpu.sync_copy(dst_hbm.at[pl.ds(base, _KC)], dstbuf)
        d1 = pltpu.async_copy(tab_hbm.at[srcbuf], srcrows, sem1)
        d2 = pltpu.async_copy(tab_hbm.at[dstbuf], dstrows, sem2)
        d1.wait()
        d2.wait()

        def grp(g, _):
            rows = g * _L + iota16
            av = plsc.load_gather(dstrows, [rows, jnp.full((_L,), 0, _i32)])
            bv = plsc.load_gather(srcrows, [rows, jnp.full((_L,), 1, _i32)])
            e = av + bv
            e = jnp.where(e > 0, e, 0.3 * e)
            ee = jnp.exp(e)
            plsc.store_scatter(outbuf, [rows, jnp.full((_L,), 0, _i32)], ee)
            for d in range(14):
                hd = plsc.load_gather(
                    srcrows, [rows, jnp.full((_L,), 2 + d, _i32)])
                plsc.store_scatter(
                    outbuf, [rows, jnp.full((_L,), 1 + d, _i32)], ee * hd)
            return 0
        lax.fori_loop(0, _KC // _L, grp, 0)
        pltpu.sync_copy(outbuf, acc.at[dstbuf], add=True)
        return 0
    lax.fori_loop(0, _EPW // _KC, chunk, 0)
    plsc.subcore_barrier()
    pltpu.sync_copy(acc.at[pl.ds(s * _RPS, _RPS)],
                    att_hbm.at[c, pl.ds(s * _RPS, _RPS)])


_sc_attn = pl.kernel(
    _sc_attn_body,
    out_type=jax.ShapeDtypeStruct((_NC, _NP, 16), _f32),
    mesh=_sc_mesh,
    compiler_params=_sc_params,
    scratch_types=[
        pltpu.VMEM((_KC,), _i32),
        pltpu.VMEM((_KC,), _i32),
        pltpu.VMEM((_KC, 16), _f32),
        pltpu.VMEM((_KC, 16), _f32),
        pltpu.VMEM((_KC, 16), _f32),
        pltpu.VMEM((_ZR, 16), _f32),
        pltpu.VMEM_SHARED((_NP, 16), _f32),
        pltpu.SemaphoreType.DMA,
        pltpu.SemaphoreType.DMA,
    ],
)


def _head_body(nv_ref, tab_ref, t0_ref, t1_ref, wqa_ref, wqb_ref, wqc_ref,
               bq1_ref, wq2_ref, bq2_ref, q_ref):
    att = t0_ref[0] + t1_ref[0]
    denom = jnp.maximum(att[:, 0:1], 1e-12)
    attd = att / denom
    y = jnp.dot(nv_ref[...], wqa_ref[...], preferred_element_type=_f32)
    y = y + jnp.dot(tab_ref[...], wqb_ref[...], preferred_element_type=_f32)
    y = y + jnp.dot(attd, wqc_ref[...], preferred_element_type=_f32)
    y = jnp.maximum(y + bq1_ref[...], 0.0)
    q = jnp.dot(y, wq2_ref[...], preferred_element_type=_f32) + bq2_ref[...]
    q_ref[...] = jnp.maximum(q, 0.0)


def _tc_head(nv, tab, att, wqa, wqb, wqc, bq1p, wq2p, bq2p):
    r = 2000
    full = lambda shape: pl.BlockSpec(shape, lambda i: tuple(0 for _ in shape))
    return pl.pallas_call(
        _head_body,
        grid=(_N // r,),
        in_specs=[
            pl.BlockSpec((r, 16), lambda i: (i, 0)),
            pl.BlockSpec((r, 16), lambda i: (i, 0)),
            pl.BlockSpec((1, r, 16), lambda i: (0, i, 0)),
            pl.BlockSpec((1, r, 16), lambda i: (1, i, 0)),
            full((16, 32)), full((16, 32)), full((16, 32)), full((1, 32)),
            full((32, 8)), full((1, 8)),
        ],
        out_specs=pl.BlockSpec((r, 8), lambda i: (i, 0)),
        out_shape=jax.ShapeDtypeStruct((_N, 8), _f32),
    )(nv, tab, att, att, wqa, wqb, wqc, bq1p, wq2p, bq2p)


def _pad2(m, r, c):
    return jnp.zeros((r, c), _f32).at[:m.shape[0], :m.shape[1]].set(m)


def _padrow(v, c):
    return jnp.zeros((1, c), _f32).at[0, :v.shape[0]].set(v)


def kernel(edge_feats, phase, edge_index, W1, b1, W2, b2, Wn, bn,
           lstm_kernel, lstm_rkernel, lstm_bias, Wa, Wq1, bq1, Wq2, bq2):
    w1p = _pad2(W1, 10, 16)
    b1p = _padrow(b1, 16)
    w2p = _pad2(W2, 16, 16)
    b2p = _padrow(b2, 16)
    wn16 = _pad2(Wn, 16, 16)
    bn16 = _padrow(bn, 16)
    ki = _pad2(lstm_kernel[:, 0:14], 16, 16)
    kc = _pad2(lstm_kernel[:, 28:42], 16, 16)
    ko = _pad2(lstm_kernel[:, 42:56], 16, 16)
    bi = _padrow(lstm_bias[0:14], 16)
    bc = _padrow(lstm_bias[28:42], 16)
    bo = _padrow(lstm_bias[42:56], 16)
    wa1 = _padrow(Wa[0:14, 0], 16)
    wa2 = _padrow(Wa[14:28, 0], 16)
    wqa = _pad2(Wq1[0:14], 16, 32)
    wqb = jnp.zeros((16, 32), _f32).at[2:16, :30].set(Wq1[14:28])
    wqc = jnp.zeros((16, 32), _f32).at[1:15, :30].set(Wq1[28:42])
    bq1p = _padrow(bq1, 32)
    wq2p = _pad2(Wq2, 32, 8)
    bq2p = _padrow(bq2, 8)

    src_idx = edge_index[0]
    dst_idx = edge_index[1]
    eh16 = _edge_mlp(edge_feats, w1p, b1p, w2p, b2p)
    agg = _sc_scatter(eh16, dst_idx)
    if isinstance(agg, (tuple, list)):
        agg = agg[0]
    nv, tab, c16 = _tc_mid(agg, phase.reshape(_N, 1), wn16, bn16,
                           ki, bi, kc, bc, ko, bo, wa1, wa2)
    att = _sc_attn(tab, src_idx, dst_idx)
    if isinstance(att, (tuple, list)):
        att = att[0]
    q8 = _tc_head(nv, tab, att, wqa, wqb, wqc, bq1p, wq2p, bq2p)
    return (q8[:, :2], tab[:, 2:16], c16[:, :14])


# speedup vs baseline: 9.1580x; 1.0379x over previous
import jax
import jax.numpy as jnp
from jax import lax
from jax.experimental import pallas as pl
from jax.experimental.pallas import tpu as pltpu
from jax.experimental.pallas import tpu_sc as plsc

_N = 100000
_E = 3200000
_NC, _NS, _L = 2, 16, 16
_NW = _NC * _NS
_KF = 512
_TOTC = _E // _KF
_NT = (_TOTC + _NW - 1) // _NW
_NP = 100352
_RPS = _NP // _NS
_ZR = 112
_f32 = jnp.float32
_i32 = jnp.int32

_sc_mesh = plsc.VectorSubcoreMesh(
    core_axis_name="c", subcore_axis_name="s", num_cores=_NC, num_subcores=_NS)
_sc_params = pltpu.CompilerParams(
    use_tc_tiling_on_sc=False, needs_layout_passes=False)


def _bf16r(v):
    u = plsc.bitcast(v, jnp.uint32)
    u = (u + jnp.uint32(0x7FFF) + ((u >> jnp.uint32(16)) & jnp.uint32(1))) \
        & jnp.uint32(0xFFFF0000)
    return plsc.bitcast(u, _f32)


def _zero_acc(zbuf, acc, s):
    zero16 = jnp.zeros((_L,), _f32)

    def zb(i, _):
        zbuf[i, :] = zero16
        return 0
    lax.fori_loop(0, _ZR, zb, 0)

    def zacc(j, _):
        pltpu.sync_copy(zbuf, acc.at[pl.ds(s * _RPS + j * _ZR, _ZR)])
        return 0
    lax.fori_loop(0, _RPS // _ZR, zacc, 0)


def _sc_mlp_scatter_body(ef_hbm, ei_hbm, w_hbm, agg_hbm,
                         ebuf, idxbuf, outbuf, zbuf, wbuf, wsplat, acc):
    c = lax.axis_index("c")
    s = lax.axis_index("s")
    wid = c * _NS + s
    iota16 = lax.iota(_i32, _L)
    iota160 = iota16 * 10
    zero16 = jnp.zeros((_L,), _f32)
    _zero_acc(zbuf, acc, s)

    pltpu.sync_copy(w_hbm, wbuf)

    for i in range(220):
        v = plsc.load_gather(wbuf, [jnp.full((_L,), i + 8, _i32)])
        if i < 100 or 110 <= i < 210:
            v = _bf16r(v)
        wsplat[i, :] = v

    def zo(i, _):
        outbuf[i, :] = zero16
        return 0
    lax.fori_loop(0, _KF, zo, 0)
    plsc.subcore_barrier()

    cconst = [jnp.full((_L,), co, _i32) for co in range(10)]

    def chunk(t, _):
        cid = t * _NW + wid

        @pl.when(cid < _TOTC)
        def _go():
            base = cid * _KF
            pltpu.sync_copy(ei_hbm.at[:, pl.ds(base, _KF)], idxbuf)
            pltpu.sync_copy(ef_hbm.at[pl.ds(base * 10, _KF * 10)], ebuf)

            def grp(gg, _2):
                r0 = gg * 32
                x = [[_bf16r(plsc.load_gather(
                    ebuf, [(r0 + u * 16) * 10 + d + iota160]))
                    for d in range(10)] for u in range(2)]
                sg = [[None] * 10 for _ in range(2)]
                for co in range(10):
                    a0 = wsplat[100 + co, :]
                    a1 = a0
                    for d in range(10):
                        w = wsplat[d * 10 + co, :]
                        a0 = a0 + x[0][d] * w
                        a1 = a1 + x[1][d] * w
                    sg[0][co] = _bf16r(1.0 / (1.0 + jnp.exp(-a0)))
                    sg[1][co] = _bf16r(1.0 / (1.0 + jnp.exp(-a1)))
                for co in range(10):
                    o0 = wsplat[210 + co, :]
                    o1 = o0
                    for d in range(10):
                        w = wsplat[110 + d * 10 + co, :]
                        o0 = o0 + sg[0][d] * w
                        o1 = o1 + sg[1][d] * w
                    o0 = jnp.maximum(o0, 0.0)
                    o1 = jnp.maximum(o1, 0.0)
                    plsc.store_scatter(outbuf, [r0 + iota16, cconst[co]], o0)
                    plsc.store_scatter(
                        outbuf, [r0 + 16 + iota16, cconst[co]], o1)
                return 0
            lax.fori_loop(0, _KF // 32, grp, 0)
            pltpu.sync_copy(outbuf, acc.at[idxbuf.at[1]], add=True)
        return 0
    lax.fori_loop(0, _NT, chunk, 0)
    plsc.subcore_barrier()
    pltpu.sync_copy(acc.at[pl.ds(s * _RPS, _RPS)],
                    agg_hbm.at[c, pl.ds(s * _RPS, _RPS)])


_sc_mlp_scatter = pl.kernel(
    _sc_mlp_scatter_body,
    out_type=jax.ShapeDtypeStruct((_NC, _NP, 16), _f32),
    mesh=_sc_mesh,
    compiler_params=_sc_params,
    scratch_types=[
        pltpu.VMEM((_KF * 10,), _f32),
        pltpu.VMEM((2, _KF), _i32),
        pltpu.VMEM((_KF, 16), _f32),
        pltpu.VMEM((_ZR, 16), _f32),
        pltpu.VMEM((232,), _f32),
        pltpu.VMEM((224, 16), _f32),
        pltpu.VMEM_SHARED((_NP, 16), _f32),
    ],
)


def _mid_body(a0_ref, a1_ref, ph_ref, wn_ref, bn_ref, ki_ref, bi_ref,
              kc_ref, bc_ref, ko_ref, bo_ref, wa1_ref, wa2_ref,
              nv_ref, tab_ref, c_ref):
    agg = a0_ref[0] + a1_ref[0]
    ph = ph_ref[...]
    lane = lax.broadcasted_iota(_i32, agg.shape, 1)
    node_in = agg + (lane == ph + 9).astype(_f32)
    nv = jnp.dot(node_in, wn_ref[...], preferred_element_type=_f32) + bn_ref[...]
    i_g = jax.nn.sigmoid(
        jnp.dot(nv, ki_ref[...], preferred_element_type=_f32) + bi_ref[...])
    c_bar = jnp.tanh(
        jnp.dot(nv, kc_ref[...], preferred_element_type=_f32) + bc_ref[...])
    o_g = jax.nn.sigmoid(
        jnp.dot(nv, ko_ref[...], preferred_element_type=_f32) + bo_ref[...])
    cc = i_g * c_bar
    h = o_g * jnp.tanh(cc)
    a = jnp.sum(h * wa1_ref[...], axis=1, keepdims=True)
    b = jnp.sum(h * wa2_ref[...], axis=1, keepdims=True)
    nv_ref[...] = nv
    c_ref[...] = cc
    tab_ref[...] = jnp.concatenate([a, b, h[:, :14]], axis=1)


def _tc_mid(agg, ph2, wn16, bn16, ki, bi, kc, bc, ko, bo, wa1, wa2):
    r = 2000
    full = lambda shape: pl.BlockSpec(shape, lambda i: tuple(0 for _ in shape))
    return pl.pallas_call(
        _mid_body,
        grid=(_N // r,),
        in_specs=[
            pl.BlockSpec((1, r, 16), lambda i: (0, i, 0)),
            pl.BlockSpec((1, r, 16), lambda i: (1, i, 0)),
            pl.BlockSpec((r, 1), lambda i: (i, 0)),
            full((16, 16)), full((1, 16)),
            full((16, 16)), full((1, 16)),
            full((16, 16)), full((1, 16)),
            full((16, 16)), full((1, 16)),
            full((1, 16)), full((1, 16)),
        ],
        out_specs=[
            pl.BlockSpec((r, 16), lambda i: (i, 0)),
            pl.BlockSpec((r, 16), lambda i: (i, 0)),
            pl.BlockSpec((r, 16), lambda i: (i, 0)),
        ],
        out_shape=[
            jax.ShapeDtypeStruct((_N, 16), _f32),
            jax.ShapeDtypeStruct((_N, 16), _f32),
            jax.ShapeDtypeStruct((_N, 16), _f32),
        ],
    )(agg, agg, ph2, wn16, bn16, ki, bi, kc, bc, ko, bo, wa1, wa2)


def _sc_attn_body(tab_hbm, ei_hbm, att_hbm, idxbuf, srcrows, dstrows,
                  outbuf, zbuf, acc, sem1, sem2):
    c = lax.axis_index("c")
    s = lax.axis_index("s")
    wid = c * _NS + s
    iota16 = lax.iota(_i32, _L)
    zero16 = jnp.zeros((_L,), _f32)
    _zero_acc(zbuf, acc, s)

    c15 = jnp.full((_L,), 15, _i32)

    def zo(g, _):
        plsc.store_scatter(outbuf, [g * _L + iota16, c15], zero16)
        return 0
    lax.fori_loop(0, _KF // _L, zo, 0)
    plsc.subcore_barrier()

    cl = [jnp.full((_L,), j, _i32) for j in range(16)]

    def chunk(t, _):
        cid = t * _NW + wid

        @pl.when(cid < _TOTC)
        def _go():
            base = cid * _KF
            pltpu.sync_copy(ei_hbm.at[:, pl.ds(base, _KF)], idxbuf)
            d1 = pltpu.async_copy(tab_hbm.at[idxbuf.at[0]], srcrows, sem1)
            d2 = pltpu.async_copy(tab_hbm.at[idxbuf.at[1]], dstrows, sem2)
            d1.wait()
            d2.wait()

            def grp(g, _2):
                rows = g * _L + iota16
                av = plsc.load_gather(dstrows, [rows, cl[0]])
                bv = plsc.load_gather(srcrows, [rows, cl[1]])
                e = av + bv
                e = jnp.where(e > 0, e, 0.3 * e)
                ee = jnp.exp(e)
                plsc.store_scatter(outbuf, [rows, cl[0]], ee)
                for d in range(14):
                    hd = plsc.load_gather(srcrows, [rows, cl[2 + d]])
                    plsc.store_scatter(outbuf, [rows, cl[1 + d]], ee * hd)
                return 0
            lax.fori_loop(0, _KF // _L, grp, 0)
            pltpu.sync_copy(outbuf, acc.at[idxbuf.at[1]], add=True)
        return 0
    lax.fori_loop(0, _NT, chunk, 0)
    plsc.subcore_barrier()
    pltpu.sync_copy(acc.at[pl.ds(s * _RPS, _RPS)],
                    att_hbm.at[c, pl.ds(s * _RPS, _RPS)])


_sc_attn = pl.kernel(
    _sc_attn_body,
    out_type=jax.ShapeDtypeStruct((_NC, _NP, 16), _f32),
    mesh=_sc_mesh,
    compiler_params=_sc_params,
    scratch_types=[
        pltpu.VMEM((2, _KF), _i32),
        pltpu.VMEM((_KF, 16), _f32),
        pltpu.VMEM((_KF, 16), _f32),
        pltpu.VMEM((_KF, 16), _f32),
        pltpu.VMEM((_ZR, 16), _f32),
        pltpu.VMEM_SHARED((_NP, 16), _f32),
        pltpu.SemaphoreType.DMA,
        pltpu.SemaphoreType.DMA,
    ],
)


def _head_body(nv_ref, tab_ref, t0_ref, t1_ref, wqa_ref, wqb_ref, wqc_ref,
               bq1_ref, wq2_ref, bq2_ref, q_ref):
    att = t0_ref[0] + t1_ref[0]
    denom = jnp.maximum(att[:, 0:1], 1e-12)
    attd = att / denom
    y = jnp.dot(nv_ref[...], wqa_ref[...], preferred_element_type=_f32)
    y = y + jnp.dot(tab_ref[...], wqb_ref[...], preferred_element_type=_f32)
    y = y + jnp.dot(attd, wqc_ref[...], preferred_element_type=_f32)
    y = jnp.maximum(y + bq1_ref[...], 0.0)
    q = jnp.dot(y, wq2_ref[...], preferred_element_type=_f32) + bq2_ref[...]
    q_ref[...] = jnp.maximum(q, 0.0)


def _tc_head(nv, tab, att, wqa, wqb, wqc, bq1p, wq2p, bq2p):
    r = 2000
    full = lambda shape: pl.BlockSpec(shape, lambda i: tuple(0 for _ in shape))
    return pl.pallas_call(
        _head_body,
        grid=(_N // r,),
        in_specs=[
            pl.BlockSpec((r, 16), lambda i: (i, 0)),
            pl.BlockSpec((r, 16), lambda i: (i, 0)),
            pl.BlockSpec((1, r, 16), lambda i: (0, i, 0)),
            pl.BlockSpec((1, r, 16), lambda i: (1, i, 0)),
            full((16, 32)), full((16, 32)), full((16, 32)), full((1, 32)),
            full((32, 8)), full((1, 8)),
        ],
        out_specs=pl.BlockSpec((r, 8), lambda i: (i, 0)),
        out_shape=jax.ShapeDtypeStruct((_N, 8), _f32),
    )(nv, tab, att, att, wqa, wqb, wqc, bq1p, wq2p, bq2p)


def _pad2(m, r, c):
    return jnp.zeros((r, c), _f32).at[:m.shape[0], :m.shape[1]].set(m)


def _padrow(v, c):
    return jnp.zeros((1, c), _f32).at[0, :v.shape[0]].set(v)


def kernel(edge_feats, phase, edge_index, W1, b1, W2, b2, Wn, bn,
           lstm_kernel, lstm_rkernel, lstm_bias, Wa, Wq1, bq1, Wq2, bq2):
    wflat = jnp.concatenate([jnp.zeros((8,), _f32), W1.reshape(-1), b1,
                             W2.reshape(-1), b2, jnp.zeros((4,), _f32)])
    efflat = edge_feats.reshape(-1)
    wn16 = _pad2(Wn, 16, 16)
    bn16 = _padrow(bn, 16)
    ki = _pad2(lstm_kernel[:, 0:14], 16, 16)
    kc = _pad2(lstm_kernel[:, 28:42], 16, 16)
    ko = _pad2(lstm_kernel[:, 42:56], 16, 16)
    bi = _padrow(lstm_bias[0:14], 16)
    bc = _padrow(lstm_bias[28:42], 16)
    bo = _padrow(lstm_bias[42:56], 16)
    wa1 = _padrow(Wa[0:14, 0], 16)
    wa2 = _padrow(Wa[14:28, 0], 16)
    wqa = _pad2(Wq1[0:14], 16, 32)
    wqb = jnp.zeros((16, 32), _f32).at[2:16, :30].set(Wq1[14:28])
    wqc = jnp.zeros((16, 32), _f32).at[1:15, :30].set(Wq1[28:42])
    bq1p = _padrow(bq1, 32)
    wq2p = _pad2(Wq2, 32, 8)
    bq2p = _padrow(bq2, 8)

    agg = _sc_mlp_scatter(efflat, edge_index, wflat)
    if isinstance(agg, (tuple, list)):
        agg = agg[0]
    nv, tab, c16 = _tc_mid(agg, phase.reshape(_N, 1), wn16, bn16,
                           ki, bi, kc, bc, ko, bo, wa1, wa2)
    att = _sc_attn(tab, edge_index)
    if isinstance(att, (tuple, list)):
        att = att[0]
    q8 = _tc_head(nv, tab, att, wqa, wqb, wqc, bq1p, wq2p, bq2p)
    return (q8[:, :2], tab[:, 2:16], c16[:, :14])
